# Initial kernel scaffold; baseline (speedup 1.0000x reference)
import jax
import jax.numpy as jnp
from jax.experimental import pallas as pl


def kernel(inputs, W_feat, b_feat, W_route, b_route, leaf_W, leaf_b):
    raise NotImplementedError("write your pallas kernel here")


# speedup vs baseline: 3.3005x; 3.3005x over previous
import functools

import jax
import jax.numpy as jnp
from jax.experimental import pallas as pl
from jax.experimental.pallas import tpu as pltpu

B, D, H, E, C = 4096, 1024, 1024, 8, 1024
LPAD = 128
TM = 512


def _feat_body(x_ref, wf_ref, bf_ref, wr_ref, br_ref, feat_ref, log_ref):
    x = x_ref[...]
    feat = jnp.dot(x, wf_ref[...], preferred_element_type=jnp.float32)
    feat_ref[...] = jnp.maximum(feat + bf_ref[...], 0.0)
    log_ref[...] = jnp.dot(x, wr_ref[...], preferred_element_type=jnp.float32) + br_ref[...]


def _dense_body(feat_ref, log_ref, lw_ref, lb_ref, out_ref):
    e = pl.program_id(1)
    logits = log_ref[...]
    lane = jax.lax.broadcasted_iota(jnp.int32, logits.shape, 1)
    valid = jnp.where(lane < E, logits, -jnp.inf)
    m = jnp.max(valid, axis=1, keepdims=True)
    choice = jnp.min(jnp.where(valid == m, lane, E), axis=1, keepdims=True)
    contrib = jnp.dot(feat_ref[...], lw_ref[0], preferred_element_type=jnp.float32)
    contrib = jnp.where(choice == e, contrib + lb_ref[0], 0.0)

    @pl.when(e == 0)
    def _():
        out_ref[...] = contrib

    @pl.when(e > 0)
    def _():
        out_ref[...] += contrib


def kernel(inputs, W_feat, b_feat, W_route, b_route, leaf_W, leaf_b):
    wr_pad = jnp.zeros((D, LPAD), W_route.dtype).at[:, :E].set(W_route)
    br_pad = jnp.zeros((1, LPAD), b_route.dtype).at[0, :E].set(b_route)

    feat, logits = pl.pallas_call(
        _feat_body,
        grid=(B // TM,),
        in_specs=[
            pl.BlockSpec((TM, D), lambda t: (t, 0)),
            pl.BlockSpec((D, H), lambda t: (0, 0)),
            pl.BlockSpec((1, H), lambda t: (0, 0)),
            pl.BlockSpec((D, LPAD), lambda t: (0, 0)),
            pl.BlockSpec((1, LPAD), lambda t: (0, 0)),
        ],
        out_specs=[
            pl.BlockSpec((TM, H), lambda t: (t, 0)),
            pl.BlockSpec((TM, LPAD), lambda t: (t, 0)),
        ],
        out_shape=[
            jax.ShapeDtypeStruct((B, H), jnp.float32),
            jax.ShapeDtypeStruct((B, LPAD), jnp.float32),
        ],
    )(inputs, W_feat, b_feat.reshape(1, H), wr_pad, br_pad)

    out = pl.pallas_call(
        _dense_body,
        grid=(B // TM, E),
        in_specs=[
            pl.BlockSpec((TM, H), lambda t, e: (t, 0)),
            pl.BlockSpec((TM, LPAD), lambda t, e: (t, 0)),
            pl.BlockSpec((1, H, C), lambda t, e: (e, 0, 0)),
            pl.BlockSpec((1, 1, C), lambda t, e: (e, 0, 0)),
        ],
        out_specs=pl.BlockSpec((TM, C), lambda t, e: (t, 0)),
        out_shape=jax.ShapeDtypeStruct((B, C), jnp.float32),
    )(feat, logits, leaf_W, leaf_b.reshape(E, 1, C))
    return out


# speedup vs baseline: 3.9242x; 1.1890x over previous
import functools

import jax
import jax.numpy as jnp
from jax import lax
from jax.experimental import pallas as pl
from jax.experimental.pallas import tpu as pltpu
from jax.experimental.pallas import tpu_sc as plsc

B, D, H, E, C = 4096, 1024, 1024, 8, 1024
LPAD = 128
NC, NS, L = 2, 16, 16
NW = NC * NS
CHUNK = B // NW
RG = CHUNK // L
TM = 256
NT = B // TM
W = NT + E - 1

def _mesh():
    return plsc.VectorSubcoreMesh(
        core_axis_name="c", subcore_axis_name="s", num_cores=NC, num_subcores=NS
    )


def _wid():
    return lax.axis_index("s") * NC + lax.axis_index("c")


def _logits_body(x_ref, wr_ref, br_ref, out_ref):
    out_ref[...] = (
        jnp.dot(x_ref[...], wr_ref[...], preferred_element_type=jnp.float32)
        + br_ref[...]
    )


def _ra_body(logits_hbm, choices_hbm, counts_hbm, log_v, ch_v, cnt_v):
    base = _wid() * CHUNK
    pltpu.sync_copy(logits_hbm.at[pl.ds(base * LPAD, CHUNK * LPAD)], log_v)
    lane = lax.iota(jnp.int32, L)
    ch_regs = []
    for j in range(RG):
        flat = (jnp.full((L,), j * L, jnp.int32) + lane) * LPAD
        best = plsc.load_gather(log_v, [flat])
        arg = jnp.zeros((L,), jnp.int32)
        for e in range(1, E):
            v = plsc.load_gather(log_v, [flat + e])
            upd = v > best
            arg = jnp.where(upd, e, arg)
            best = jnp.where(upd, v, best)
        ch_regs.append(arg)
        ch_v[pl.ds(j * L, L)] = arg
    cnts = jnp.zeros((L,), jnp.int32)
    for j in range(RG):
        for e in range(E):
            c = plsc.all_reduce_population_count(ch_regs[j] == e)
            cnts = cnts + jnp.where(lane == e, c, 0)
    cnt_v[...] = cnts
    pltpu.sync_copy(ch_v, choices_hbm.at[pl.ds(base, CHUNK)])
    pltpu.sync_copy(cnt_v, counts_hbm.at[_wid()])


_SCAT = 32
_NB = CHUNK // _SCAT


def _rb_body(choices_hbm, counts_hbm, x_hbm,
             inv_hbm, sx_hbm, offs_hbm,
             ch_v, cnts_v, inv_v, offs_v, idx0, idx1, idx2, idx3,
             xbuf, sem):
    wid = _wid()
    base = wid * CHUNK
    pltpu.sync_copy(choices_hbm.at[pl.ds(base, CHUNK)], ch_v)
    pltpu.sync_copy(counts_hbm, cnts_v)
    lane = lax.iota(jnp.int32, L)
    totals = jnp.zeros((L,), jnp.int32)
    prefix = jnp.zeros((L,), jnp.int32)
    for w in range(NW):
        row = cnts_v[w, :]
        totals = totals + row
        prefix = prefix + jnp.where(w < wid, row, 0)
    ebase = plsc.cumsum(totals) - totals

    @pl.when(wid == 0)
    def _():
        offs_v[...] = ebase
        pltpu.sync_copy(offs_v, offs_hbm)

    running = ebase + prefix
    idx_bufs = (idx0, idx1, idx2, idx3)
    for j in range(RG):
        v = ch_v[pl.ds(j * L, L)]
        dest = jnp.zeros((L,), jnp.int32)
        for e in range(E):
            m = v == e
            ones = m.astype(jnp.int32)
            rank = plsc.cumsum(ones) - 1
            base_e = jnp.sum(jnp.where(lane == e, running, 0))
            dest = jnp.where(m, base_e + rank, dest)
            c = plsc.all_reduce_population_count(m)
            running = running + jnp.where(lane == e, c, 0)
        inv_v[pl.ds(j * L, L)] = dest
        idx_bufs[j // 2][pl.ds((j % 2) * L, L)] = dest
    pltpu.sync_copy(inv_v, inv_hbm.at[pl.ds(base, CHUNK)])
    for k in range(_NB):
        pltpu.sync_copy(x_hbm.at[pl.ds(base + k * _SCAT, _SCAT)], xbuf)
        pltpu.async_copy(xbuf, sx_hbm.at[idx_bufs[k]], sem).wait()


def _g_body(tm_r, em_r, vl_r, of_r,
            xs_ref, wf_ref, bf_ref, lw_ref, lb_ref, out_ref, feat_scr):
    w = pl.program_id(0)
    t = tm_r[w]
    e = em_r[w]
    prev_t = tm_r[jnp.maximum(w - 1, 0)]
    first = jnp.logical_or(w == 0, t != prev_t)

    @pl.when(first)
    def _():
        f = jnp.dot(xs_ref[...], wf_ref[...], preferred_element_type=jnp.float32)
        feat_scr[...] = jnp.maximum(f + bf_ref[...], 0.0)

    p = t * TM + lax.broadcasted_iota(jnp.int32, (TM, 1), 0)
    mask = (p >= of_r[e]) & (p < of_r[e + 1]) & (vl_r[w] == 1)
    contrib = jnp.dot(feat_scr[...], lw_ref[0], preferred_element_type=jnp.float32)
    contrib = jnp.where(mask, contrib + lb_ref[0], 0.0)

    @pl.when(first)
    def _():
        out_ref[...] = contrib

    @pl.when(jnp.logical_not(first))
    def _():
        out_ref[...] += contrib


def _u_body(so_hbm, inv_hbm, out_hbm, idx_v, buf, sem):
    base = _wid() * CHUNK
    for k in range(_NB):
        pltpu.sync_copy(inv_hbm.at[pl.ds(base + k * _SCAT, _SCAT)], idx_v)
        pltpu.async_copy(so_hbm.at[idx_v], buf, sem).wait()
        pltpu.sync_copy(buf, out_hbm.at[pl.ds(base + k * _SCAT, _SCAT)])


def kernel(inputs, W_feat, b_feat, W_route, b_route, leaf_W, leaf_b):
    wr_pad = jnp.zeros((D, LPAD), W_route.dtype).at[:, :E].set(W_route)
    br_pad = jnp.zeros((1, LPAD), b_route.dtype).at[0, :E].set(b_route)

    logits = pl.pallas_call(
        _logits_body,
        grid=(4,),
        in_specs=[
            pl.BlockSpec((B // 4, D), lambda i: (i, 0)),
            pl.BlockSpec((D, LPAD), lambda i: (0, 0)),
            pl.BlockSpec((1, LPAD), lambda i: (0, 0)),
        ],
        out_specs=pl.BlockSpec((B // 4, LPAD), lambda i: (i, 0)),
        out_shape=jax.ShapeDtypeStruct((B, LPAD), jnp.float32),
    )(inputs, wr_pad, br_pad)

    choices, counts = pl.kernel(
        _ra_body,
        out_type=[
            jax.ShapeDtypeStruct((B,), jnp.int32),
            jax.ShapeDtypeStruct((NW, L), jnp.int32),
        ],
        mesh=_mesh(),
        compiler_params=pltpu.CompilerParams(needs_layout_passes=False),
        scratch_types=[
            pltpu.VMEM((CHUNK * LPAD,), jnp.float32),
            pltpu.VMEM((CHUNK,), jnp.int32),
            pltpu.VMEM((L,), jnp.int32),
        ],
    )(logits.reshape(-1))

    inv, sorted_x, offs16 = pl.kernel(
        _rb_body,
        out_type=[
            jax.ShapeDtypeStruct((B,), jnp.int32),
            jax.ShapeDtypeStruct((B, D), jnp.float32),
            jax.ShapeDtypeStruct((L,), jnp.int32),
        ],
        mesh=_mesh(),
        compiler_params=pltpu.CompilerParams(needs_layout_passes=False),
        scratch_types=[
            pltpu.VMEM((CHUNK,), jnp.int32),
            pltpu.VMEM((NW, L), jnp.int32),
            pltpu.VMEM((CHUNK,), jnp.int32),
            pltpu.VMEM((L,), jnp.int32),
            pltpu.VMEM((_SCAT,), jnp.int32),
            pltpu.VMEM((_SCAT,), jnp.int32),
            pltpu.VMEM((_SCAT,), jnp.int32),
            pltpu.VMEM((_SCAT,), jnp.int32),
            pltpu.VMEM((_SCAT, D), jnp.float32),
            pltpu.SemaphoreType.DMA,
        ],
    )(choices, counts, inputs)

    offs = offs16[:E + 1]
    t_arr = jnp.arange(NT, dtype=jnp.int32)
    e_lo = (jnp.searchsorted(offs, t_arr * TM, side="right") - 1).astype(jnp.int32)
    e_hi = (jnp.searchsorted(offs, t_arr * TM + (TM - 1), side="right") - 1).astype(jnp.int32)
    cnt = e_hi - e_lo + 1
    starts = jnp.cumsum(cnt) - cnt
    n_items = starts[-1] + cnt[-1]
    w_arr = jnp.arange(W, dtype=jnp.int32)
    t_map = (jnp.searchsorted(starts, w_arr, side="right") - 1).astype(jnp.int32)
    e_map = jnp.minimum(e_lo[t_map] + (w_arr - starts[t_map]), e_hi[t_map]).astype(jnp.int32)
    valid = (w_arr < n_items).astype(jnp.int32)

    sorted_out = pl.pallas_call(
        _g_body,
        grid_spec=pltpu.PrefetchScalarGridSpec(
            num_scalar_prefetch=4,
            grid=(W,),
            in_specs=[
                pl.BlockSpec((TM, D), lambda w, tm, em, vl, of: (tm[w], 0)),
                pl.BlockSpec((D, H), lambda w, tm, em, vl, of: (0, 0)),
                pl.BlockSpec((1, H), lambda w, tm, em, vl, of: (0, 0)),
                pl.BlockSpec((1, H, C), lambda w, tm, em, vl, of: (em[w], 0, 0)),
                pl.BlockSpec((1, 1, C), lambda w, tm, em, vl, of: (em[w], 0, 0)),
            ],
            out_specs=pl.BlockSpec((TM, C), lambda w, tm, em, vl, of: (tm[w], 0)),
            scratch_shapes=[pltpu.VMEM((TM, H), jnp.float32)],
        ),
        out_shape=jax.ShapeDtypeStruct((B, C), jnp.float32),
    )(t_map, e_map, valid, offs, sorted_x, W_feat, b_feat.reshape(1, H),
      leaf_W, leaf_b.reshape(E, 1, C))

    predictions = pl.kernel(
        _u_body,
        out_type=jax.ShapeDtypeStruct((B, C), jnp.float32),
        mesh=_mesh(),
        compiler_params=pltpu.CompilerParams(needs_layout_passes=False),
        scratch_types=[
            pltpu.VMEM((_SCAT,), jnp.int32),
            pltpu.VMEM((_SCAT, C), jnp.float32),
            pltpu.SemaphoreType.DMA,
        ],
    )(sorted_out, inv)
    return predictions


# speedup vs baseline: 4.4332x; 1.1297x over previous
import functools

import jax
import jax.numpy as jnp
from jax import lax
from jax.experimental import pallas as pl
from jax.experimental.pallas import tpu as pltpu
from jax.experimental.pallas import tpu_sc as plsc

B, D, H, E, C = 4096, 1024, 1024, 8, 1024
NC, NS, L = 2, 16, 16
NW = NC * NS
CHUNK = B // NW
RG = CHUNK // L
TM = 256
NT = B // TM
W = NT + E - 1
WK = 6 * L
assert NT == L


def _mesh():
    return plsc.VectorSubcoreMesh(
        core_axis_name="c", subcore_axis_name="s", num_cores=NC, num_subcores=NS
    )


def _wid():
    return lax.axis_index("s") * NC + lax.axis_index("c")


def _logits_body(x_ref, wr_ref, br_ref, out_ref):
    out_ref[...] = (
        jnp.dot(x_ref[...], wr_ref[...], preferred_element_type=jnp.float32)
        + br_ref[...]
    )


def _ra_body(logits_hbm, choices_hbm, counts_hbm, log_v, ch_v, cnt_v):
    base = _wid() * CHUNK
    pltpu.sync_copy(logits_hbm.at[pl.ds(base * E, CHUNK * E)], log_v)
    lane = lax.iota(jnp.int32, L)
    ch_regs = []
    for j in range(RG):
        flat = (jnp.full((L,), j * L, jnp.int32) + lane) * E
        best = plsc.load_gather(log_v, [flat])
        arg = jnp.zeros((L,), jnp.int32)
        for e in range(1, E):
            v = plsc.load_gather(log_v, [flat + e])
            upd = v > best
            arg = jnp.where(upd, e, arg)
            best = jnp.where(upd, v, best)
        ch_regs.append(arg)
        ch_v[pl.ds(j * L, L)] = arg
    cnts = jnp.zeros((L,), jnp.int32)
    for j in range(RG):
        for e in range(E):
            c = plsc.all_reduce_population_count(ch_regs[j] == e)
            cnts = cnts + jnp.where(lane == e, c, 0)
    cnt_v[...] = cnts
    pltpu.sync_copy(ch_v, choices_hbm.at[pl.ds(base, CHUNK)])
    pltpu.sync_copy(cnt_v, counts_hbm.at[_wid()])


_SCAT = 32
_NB = CHUNK // _SCAT


def _rb_body(choices_hbm, counts_hbm, x_hbm,
             inv_hbm, sx_hbm, wk_hbm,
             ch_v, cnts_v, inv_v, wk_v, idx0, idx1, idx2, idx3,
             xbuf, sem):
    wid = _wid()
    base = wid * CHUNK
    pltpu.sync_copy(choices_hbm.at[pl.ds(base, CHUNK)], ch_v)
    pltpu.sync_copy(counts_hbm, cnts_v)
    lane = lax.iota(jnp.int32, L)
    totals = jnp.zeros((L,), jnp.int32)
    prefix = jnp.zeros((L,), jnp.int32)
    for w in range(NW):
        row = cnts_v[w, :]
        totals = totals + row
        prefix = prefix + jnp.where(w < wid, row, 0)
    ebase = plsc.cumsum(totals) - totals

    running = ebase + prefix
    idx_bufs = (idx0, idx1, idx2, idx3)
    for j in range(RG):
        v = ch_v[pl.ds(j * L, L)]
        dest = jnp.zeros((L,), jnp.int32)
        for e in range(E):
            m = v == e
            ones = m.astype(jnp.int32)
            rank = plsc.cumsum(ones) - 1
            base_e = jnp.sum(jnp.where(lane == e, running, 0))
            dest = jnp.where(m, base_e + rank, dest)
            c = plsc.all_reduce_population_count(m)
            running = running + jnp.where(lane == e, c, 0)
        inv_v[pl.ds(j * L, L)] = dest
        idx_bufs[j // 2][pl.ds((j % 2) * L, L)] = dest
    pltpu.sync_copy(inv_v, inv_hbm.at[pl.ds(base, CHUNK)])
    for k in range(_NB):
        pltpu.sync_copy(x_hbm.at[pl.ds(base + k * _SCAT, _SCAT)], xbuf)
        pltpu.async_copy(xbuf, sx_hbm.at[idx_bufs[k]], sem).wait()

    @pl.when(wid == 0)
    def _():
        tstart = lane * TM
        e_lo = jnp.full((L,), -1, jnp.int32)
        e_hi = jnp.full((L,), -1, jnp.int32)
        for e in range(E + 1):
            off_e = jnp.sum(jnp.where(lane == e, ebase, 0))
            e_lo = e_lo + (off_e <= tstart).astype(jnp.int32)
            e_hi = e_hi + (off_e <= tstart + (TM - 1)).astype(jnp.int32)
        cntv = e_hi - e_lo + 1
        startsv = plsc.cumsum(cntv) - cntv
        n_items = jnp.sum(cntv)
        for half in range(2):
            wv = lane + half * L
            tmap = jnp.full((L,), -1, jnp.int32)
            for t in range(NT):
                s_t = jnp.sum(jnp.where(lane == t, startsv, 0))
                tmap = tmap + (s_t <= wv).astype(jnp.int32)
            emap = jnp.zeros((L,), jnp.int32)
            for t in range(NT):
                s_t = jnp.sum(jnp.where(lane == t, startsv, 0))
                lo_t = jnp.sum(jnp.where(lane == t, e_lo, 0))
                hi_t = jnp.sum(jnp.where(lane == t, e_hi, 0))
                emap = jnp.where(tmap == t,
                                 jnp.minimum(lo_t + (wv - s_t), hi_t), emap)
            wk_v[pl.ds(half * L, L)] = tmap
            wk_v[pl.ds(2 * L + half * L, L)] = emap
        wk_v[pl.ds(4 * L, L)] = ebase
        wk_v[pl.ds(5 * L, L)] = jnp.zeros((L,), jnp.int32) + n_items
        pltpu.sync_copy(wk_v, wk_hbm)


def _g_body(wk_r, xs_ref, wf_ref, bf_ref, lw_ref, lb_ref, out_ref, feat_scr):
    w = pl.program_id(0)
    t = wk_r[w]
    e = wk_r[2 * L + w]
    prev_t = wk_r[jnp.maximum(w - 1, 0)]
    first = jnp.logical_or(w == 0, t != prev_t)

    @pl.when(first)
    def _():
        f = jnp.dot(xs_ref[...], wf_ref[...], preferred_element_type=jnp.float32)
        feat_scr[...] = jnp.maximum(f + bf_ref[...], 0.0)

    p = t * TM + lax.broadcasted_iota(jnp.int32, (TM, 1), 0)
    mask = (p >= wk_r[4 * L + e]) & (p < wk_r[4 * L + e + 1]) & (w < wk_r[5 * L])
    contrib = jnp.dot(feat_scr[...], lw_ref[0], preferred_element_type=jnp.float32)
    contrib = jnp.where(mask, contrib + lb_ref[0], 0.0)

    @pl.when(first)
    def _():
        out_ref[...] = contrib

    @pl.when(jnp.logical_not(first))
    def _():
        out_ref[...] += contrib


def _u_body(so_hbm, inv_hbm, out_hbm, idx_v, buf, sem):
    base = _wid() * CHUNK
    for k in range(_NB):
        pltpu.sync_copy(inv_hbm.at[pl.ds(base + k * _SCAT, _SCAT)], idx_v)
        pltpu.async_copy(so_hbm.at[idx_v], buf, sem).wait()
        pltpu.sync_copy(buf, out_hbm.at[pl.ds(base + k * _SCAT, _SCAT)])


def kernel(inputs, W_feat, b_feat, W_route, b_route, leaf_W, leaf_b):
    logits = pl.pallas_call(
        _logits_body,
        grid=(4,),
        in_specs=[
            pl.BlockSpec((B // 4, D), lambda i: (i, 0)),
            pl.BlockSpec((D, E), lambda i: (0, 0)),
            pl.BlockSpec((1, E), lambda i: (0, 0)),
        ],
        out_specs=pl.BlockSpec((B // 4, E), lambda i: (i, 0)),
        out_shape=jax.ShapeDtypeStruct((B, E), jnp.float32),
    )(inputs, W_route, b_route.reshape(1, E))

    choices, counts = pl.kernel(
        _ra_body,
        out_type=[
            jax.ShapeDtypeStruct((B,), jnp.int32),
            jax.ShapeDtypeStruct((NW, L), jnp.int32),
        ],
        mesh=_mesh(),
        compiler_params=pltpu.CompilerParams(needs_layout_passes=False),
        scratch_types=[
            pltpu.VMEM((CHUNK * E,), jnp.float32),
            pltpu.VMEM((CHUNK,), jnp.int32),
            pltpu.VMEM((L,), jnp.int32),
        ],
    )(logits.reshape(-1))

    inv, sorted_x, wk = pl.kernel(
        _rb_body,
        out_type=[
            jax.ShapeDtypeStruct((B,), jnp.int32),
            jax.ShapeDtypeStruct((B, D), jnp.float32),
            jax.ShapeDtypeStruct((WK,), jnp.int32),
        ],
        mesh=_mesh(),
        compiler_params=pltpu.CompilerParams(needs_layout_passes=False),
        scratch_types=[
            pltpu.VMEM((CHUNK,), jnp.int32),
            pltpu.VMEM((NW, L), jnp.int32),
            pltpu.VMEM((CHUNK,), jnp.int32),
            pltpu.VMEM((WK,), jnp.int32),
            pltpu.VMEM((_SCAT,), jnp.int32),
            pltpu.VMEM((_SCAT,), jnp.int32),
            pltpu.VMEM((_SCAT,), jnp.int32),
            pltpu.VMEM((_SCAT,), jnp.int32),
            pltpu.VMEM((_SCAT, D), jnp.float32),
            pltpu.SemaphoreType.DMA,
        ],
    )(choices, counts, inputs)

    sorted_out = pl.pallas_call(
        _g_body,
        grid_spec=pltpu.PrefetchScalarGridSpec(
            num_scalar_prefetch=1,
            grid=(W,),
            in_specs=[
                pl.BlockSpec((TM, D), lambda w, wk: (wk[w], 0)),
                pl.BlockSpec((D, H), lambda w, wk: (0, 0)),
                pl.BlockSpec((1, H), lambda w, wk: (0, 0)),
                pl.BlockSpec((1, H, C), lambda w, wk: (wk[2 * L + w], 0, 0)),
                pl.BlockSpec((1, 1, C), lambda w, wk: (wk[2 * L + w], 0, 0)),
            ],
            out_specs=pl.BlockSpec((TM, C), lambda w, wk: (wk[w], 0)),
            scratch_shapes=[pltpu.VMEM((TM, H), jnp.float32)],
        ),
        out_shape=jax.ShapeDtypeStruct((B, C), jnp.float32),
    )(wk, sorted_x, W_feat, b_feat.reshape(1, H),
      leaf_W, leaf_b.reshape(E, 1, C))

    predictions = pl.kernel(
        _u_body,
        out_type=jax.ShapeDtypeStruct((B, C), jnp.float32),
        mesh=_mesh(),
        compiler_params=pltpu.CompilerParams(needs_layout_passes=False),
        scratch_types=[
            pltpu.VMEM((_SCAT,), jnp.int32),
            pltpu.VMEM((_SCAT, C), jnp.float32),
            pltpu.SemaphoreType.DMA,
        ],
    )(sorted_out, inv)
    return predictions
